# baseline (speedup 1.0000x reference)
import jax
import jax.numpy as jnp
from jax.experimental import pallas as pl

P = 4096
B = 6
F_IN = 960
PACK = 8
FP = F_IN * PACK
PR = P // PACK
BP1 = 256
BP2 = 512


def _stage1_body(emb_ref, refp_ref, w_emb_ref, w_refp_ref, b96_ref, out_ref):
    x = emb_ref[0, :, :]
    s = jnp.dot(x, w_emb_ref[:, :], preferred_element_type=jnp.float32)
    s = s + jnp.dot(refp_ref[:, :], w_refp_ref[:, :],
                    preferred_element_type=jnp.float32)
    out_ref[0, :, :] = s + b96_ref[0:1, :]


def _stage2_body(adj_ref, s36_ref, sself_ref, pm_ref, rgb_ref):
    res = jnp.dot(adj_ref[:, :], s36_ref[:, :],
                  preferred_element_type=jnp.float32)
    res = res + sself_ref[:, :]
    pm_ref[:, :] = jnp.tanh(res[:, 0:18])
    rgb_ref[:, :] = jax.nn.sigmoid(res[:, 18:36])


def kernel(embeddings, ref, adj, W_d, Wl_d, b_d, W_r, Wl_r, b_r):
    f32 = jnp.float32
    W12 = jnp.concatenate([W_d, W_r, Wl_d, Wl_r], axis=1).astype(f32)
    eye8 = jnp.eye(PACK, dtype=f32)
    w_emb = jnp.kron(eye8, W12[:F_IN, :])
    w_ref3 = jnp.pad(W12[F_IN:, :], ((0, 5), (0, 0)))
    w_refp = jnp.kron(eye8, w_ref3)
    refp = jnp.pad(ref[0].astype(f32), ((0, 0), (0, 5)))
    refp2 = refp.reshape(PR, PACK * 8)
    emb2 = embeddings.astype(f32).reshape(B, PR, FP)
    b12 = jnp.concatenate([jnp.zeros((6,), f32), b_d.astype(f32),
                           b_r.astype(f32)])
    b96 = jnp.tile(b12, (8, PACK))

    nb1 = PR // BP1
    s_pack = pl.pallas_call(
        _stage1_body,
        grid=(B, nb1),
        in_specs=[
            pl.BlockSpec((1, BP1, FP), lambda b, i: (b, i, 0)),
            pl.BlockSpec((BP1, PACK * 8), lambda b, i: (i, 0)),
            pl.BlockSpec((FP, PACK * 12), lambda b, i: (0, 0)),
            pl.BlockSpec((PACK * 8, PACK * 12), lambda b, i: (0, 0)),
            pl.BlockSpec((8, PACK * 12), lambda b, i: (0, 0)),
        ],
        out_specs=pl.BlockSpec((1, BP1, PACK * 12), lambda b, i: (b, i, 0)),
        out_shape=jax.ShapeDtypeStruct((B, PR, PACK * 12), f32),
    )(emb2, refp2, w_emb, w_refp, b96)

    s_all = s_pack.reshape(B, P, 12)
    sd = s_all[:, :, 0:3].transpose(1, 0, 2).reshape(P, 18)
    sr = s_all[:, :, 3:6].transpose(1, 0, 2).reshape(P, 18)
    s36 = jnp.concatenate([sd, sr], axis=1)
    ld = s_all[:, :, 6:9].transpose(1, 0, 2).reshape(P, 18)
    lr = s_all[:, :, 9:12].transpose(1, 0, 2).reshape(P, 18)
    sself = jnp.concatenate([ld, lr], axis=1)

    nb2 = P // BP2
    pm18, rgb18 = pl.pallas_call(
        _stage2_body,
        grid=(nb2,),
        in_specs=[
            pl.BlockSpec((BP2, P), lambda j: (j, 0)),
            pl.BlockSpec((P, 36), lambda j: (0, 0)),
            pl.BlockSpec((BP2, 36), lambda j: (j, 0)),
        ],
        out_specs=[
            pl.BlockSpec((BP2, 18), lambda j: (j, 0)),
            pl.BlockSpec((BP2, 18), lambda j: (j, 0)),
        ],
        out_shape=[
            jax.ShapeDtypeStruct((P, 18), f32),
            jax.ShapeDtypeStruct((P, 18), f32),
        ],
    )(adj.astype(f32), s36, sself)

    points_move = pm18.reshape(P, B, 3).transpose(1, 0, 2)
    rgb = rgb18.reshape(P, B, 3).transpose(1, 0, 2)
    return (points_move, rgb)


# speedup vs baseline: 1.4525x; 1.4525x over previous
import jax
import jax.numpy as jnp
from jax.experimental import pallas as pl

P = 4096
B = 6
F_IN = 960
NW = 4
BQ = 512
BP2 = 512


def _stage1_body(e0_ref, e1_ref, e2_ref, e3_ref, refp_ref, w_emb_ref,
                 w_refp_ref, b12_ref, out_ref):
    x = jnp.concatenate([e0_ref[:, :], e1_ref[:, :],
                         e2_ref[:, :], e3_ref[:, :]], axis=0)
    s = jnp.dot(x, w_emb_ref[:, :], preferred_element_type=jnp.float32)
    s = s + jnp.dot(refp_ref[:, :], w_refp_ref[:, :],
                    preferred_element_type=jnp.float32)
    out_ref[:, :] = s + b12_ref[0:1, :]


def _stage2_body(adj_ref, s36_ref, sself_ref, pm_ref, rgb_ref):
    res = jnp.dot(adj_ref[:, :], s36_ref[:, :],
                  preferred_element_type=jnp.float32)
    res = res + sself_ref[:, :]
    pm_ref[:, :] = jnp.tanh(res[:, 0:18])
    rgb_ref[:, :] = jax.nn.sigmoid(res[:, 18:36])


def kernel(embeddings, ref, adj, W_d, Wl_d, b_d, W_r, Wl_r, b_r):
    f32 = jnp.float32
    W12 = jnp.concatenate([W_d, W_r, Wl_d, Wl_r], axis=1).astype(f32)
    w_emb = W12[:F_IN, :]
    w_refp = jnp.pad(W12[F_IN:, :], ((0, 5), (0, 0)))
    refp = jnp.pad(ref[0].astype(f32), ((0, 0), (0, 5)))
    b12 = jnp.concatenate([jnp.zeros((6,), f32), b_d.astype(f32),
                           b_r.astype(f32)])
    b12 = jnp.tile(b12[None, :], (8, 1))

    emb2d = embeddings.reshape(B * P, F_IN)
    nrb = P // BQ
    grid1 = (B * P) // (NW * BQ)

    def _estream(w):
        return pl.BlockSpec((BQ, F_IN), lambda g, w=w: (g * NW + w, 0))

    s_flat = pl.pallas_call(
        _stage1_body,
        grid=(grid1,),
        in_specs=[
            _estream(0), _estream(1), _estream(2), _estream(3),
            pl.BlockSpec((NW * BQ, 8), lambda g: (g % (nrb // NW), 0)),
            pl.BlockSpec((F_IN, 12), lambda g: (0, 0)),
            pl.BlockSpec((8, 12), lambda g: (0, 0)),
            pl.BlockSpec((8, 12), lambda g: (0, 0)),
        ],
        out_specs=pl.BlockSpec((NW * BQ, 12), lambda g: (g, 0)),
        out_shape=jax.ShapeDtypeStruct((B * P, 12), f32),
    )(emb2d, emb2d, emb2d, emb2d, refp, w_emb, w_refp, b12)

    s_all = s_flat.reshape(B, P, 12)
    sd = s_all[:, :, 0:3].transpose(1, 0, 2).reshape(P, 18)
    sr = s_all[:, :, 3:6].transpose(1, 0, 2).reshape(P, 18)
    s36 = jnp.concatenate([sd, sr], axis=1)
    ld = s_all[:, :, 6:9].transpose(1, 0, 2).reshape(P, 18)
    lr = s_all[:, :, 9:12].transpose(1, 0, 2).reshape(P, 18)
    sself = jnp.concatenate([ld, lr], axis=1)

    nb2 = P // BP2
    pm18, rgb18 = pl.pallas_call(
        _stage2_body,
        grid=(nb2,),
        in_specs=[
            pl.BlockSpec((BP2, P), lambda j: (j, 0)),
            pl.BlockSpec((P, 36), lambda j: (0, 0)),
            pl.BlockSpec((BP2, 36), lambda j: (j, 0)),
        ],
        out_specs=[
            pl.BlockSpec((BP2, 18), lambda j: (j, 0)),
            pl.BlockSpec((BP2, 18), lambda j: (j, 0)),
        ],
        out_shape=[
            jax.ShapeDtypeStruct((P, 18), f32),
            jax.ShapeDtypeStruct((P, 18), f32),
        ],
    )(adj.astype(f32), s36, sself)

    points_move = pm18.reshape(P, B, 3).transpose(1, 0, 2)
    rgb = rgb18.reshape(P, B, 3).transpose(1, 0, 2)
    return (points_move, rgb)


# speedup vs baseline: 1.4533x; 1.0005x over previous
import jax
import jax.numpy as jnp
from jax.experimental import pallas as pl

P = 4096
B = 6
F_IN = 960
NW = 4
BQ = 512
BP2 = 512


def _stage1_body(e0_ref, e1_ref, e2_ref, e3_ref, refp_ref, w_emb_ref,
                 w_refp_ref, b12_ref, out_ref):
    rp = jnp.dot(refp_ref[:, :], w_refp_ref[:, :],
                 preferred_element_type=jnp.float32) + b12_ref[0:1, :]
    for w, e_ref in enumerate((e0_ref, e1_ref, e2_ref, e3_ref)):
        s = jnp.dot(e_ref[:, :], w_emb_ref[:, :],
                    preferred_element_type=jnp.float32)
        out_ref[w * BQ:(w + 1) * BQ, :] = s + rp[w * BQ:(w + 1) * BQ, :]


def _stage2_body(adj_ref, s36_ref, sself_ref, pm_ref, rgb_ref):
    res = jnp.dot(adj_ref[:, :], s36_ref[:, :],
                  preferred_element_type=jnp.float32)
    res = res + sself_ref[:, :]
    pm_ref[:, :] = jnp.tanh(res[:, 0:18])
    rgb_ref[:, :] = jax.nn.sigmoid(res[:, 18:36])


def kernel(embeddings, ref, adj, W_d, Wl_d, b_d, W_r, Wl_r, b_r):
    f32 = jnp.float32
    W12 = jnp.concatenate([W_d, W_r, Wl_d, Wl_r], axis=1).astype(f32)
    w_emb = W12[:F_IN, :]
    w_refp = jnp.pad(W12[F_IN:, :], ((0, 5), (0, 0)))
    refp = jnp.pad(ref[0].astype(f32), ((0, 0), (0, 5)))
    b12 = jnp.concatenate([jnp.zeros((6,), f32), b_d.astype(f32),
                           b_r.astype(f32)])
    b12 = jnp.tile(b12[None, :], (8, 1))

    emb2d = embeddings.reshape(B * P, F_IN)
    nrb = P // BQ
    grid1 = (B * P) // (NW * BQ)

    def _estream(w):
        return pl.BlockSpec((BQ, F_IN), lambda g, w=w: (g * NW + w, 0))

    s_flat = pl.pallas_call(
        _stage1_body,
        grid=(grid1,),
        in_specs=[
            _estream(0), _estream(1), _estream(2), _estream(3),
            pl.BlockSpec((NW * BQ, 8), lambda g: (g % (nrb // NW), 0)),
            pl.BlockSpec((F_IN, 12), lambda g: (0, 0)),
            pl.BlockSpec((8, 12), lambda g: (0, 0)),
            pl.BlockSpec((8, 12), lambda g: (0, 0)),
        ],
        out_specs=pl.BlockSpec((NW * BQ, 12), lambda g: (g, 0)),
        out_shape=jax.ShapeDtypeStruct((B * P, 12), f32),
    )(emb2d, emb2d, emb2d, emb2d, refp, w_emb, w_refp, b12)

    s_all = s_flat.reshape(B, P, 12)
    sd = s_all[:, :, 0:3].transpose(1, 0, 2).reshape(P, 18)
    sr = s_all[:, :, 3:6].transpose(1, 0, 2).reshape(P, 18)
    s36 = jnp.concatenate([sd, sr], axis=1)
    ld = s_all[:, :, 6:9].transpose(1, 0, 2).reshape(P, 18)
    lr = s_all[:, :, 9:12].transpose(1, 0, 2).reshape(P, 18)
    sself = jnp.concatenate([ld, lr], axis=1)

    nb2 = P // BP2
    pm18, rgb18 = pl.pallas_call(
        _stage2_body,
        grid=(nb2,),
        in_specs=[
            pl.BlockSpec((BP2, P), lambda j: (j, 0)),
            pl.BlockSpec((P, 36), lambda j: (0, 0)),
            pl.BlockSpec((BP2, 36), lambda j: (j, 0)),
        ],
        out_specs=[
            pl.BlockSpec((BP2, 18), lambda j: (j, 0)),
            pl.BlockSpec((BP2, 18), lambda j: (j, 0)),
        ],
        out_shape=[
            jax.ShapeDtypeStruct((P, 18), f32),
            jax.ShapeDtypeStruct((P, 18), f32),
        ],
    )(adj.astype(f32), s36, sself)

    points_move = pm18.reshape(P, B, 3).transpose(1, 0, 2)
    rgb = rgb18.reshape(P, B, 3).transpose(1, 0, 2)
    return (points_move, rgb)


# speedup vs baseline: 2.0559x; 1.4147x over previous
import jax
import jax.numpy as jnp
from jax.experimental import pallas as pl

P = 4096
B = 6
F_IN = 960
NW = 4
BQ = 512
BP2 = 512


def _stage1_body(e0_ref, e1_ref, e2_ref, e3_ref, refp_ref, w_emb_ref,
                 w_refp_ref, b12_ref, out_ref):
    rp = jax.lax.dot_general(w_refp_ref[:, :], refp_ref[:, :],
                             dimension_numbers=((([0]), ([1])), ((), ())),
                             preferred_element_type=jnp.float32)
    rp = rp + b12_ref[:, 0:1]
    for w, e_ref in enumerate((e0_ref, e1_ref, e2_ref, e3_ref)):
        s = jax.lax.dot_general(w_emb_ref[:, :], e_ref[:, :],
                                dimension_numbers=((([0]), ([1])), ((), ())),
                                preferred_element_type=jnp.float32)
        out_ref[:, w * BQ:(w + 1) * BQ] = s + rp[:, w * BQ:(w + 1) * BQ]


def _stage2_body(adj_ref, s36_ref, sself_ref, pm_ref, rgb_ref):
    res = jnp.dot(adj_ref[:, :], s36_ref[:, :],
                  preferred_element_type=jnp.float32)
    res = res + sself_ref[:, :]
    pm_ref[:, :] = jnp.tanh(res[:, 0:18])
    rgb_ref[:, :] = jax.nn.sigmoid(res[:, 18:36])


def kernel(embeddings, ref, adj, W_d, Wl_d, b_d, W_r, Wl_r, b_r):
    f32 = jnp.float32
    W12 = jnp.concatenate([W_d, W_r, Wl_d, Wl_r], axis=1).astype(f32)
    w_emb = W12[:F_IN, :]
    w_refp = jnp.pad(W12[F_IN:, :], ((0, 5), (0, 0)))
    refp = jnp.pad(ref[0].astype(f32), ((0, 0), (0, 5)))
    b12 = jnp.concatenate([jnp.zeros((6,), f32), b_d.astype(f32),
                           b_r.astype(f32)])
    b12t = jnp.tile(b12[:, None], (1, 128))

    emb2d = embeddings.reshape(B * P, F_IN)
    nrb = P // BQ
    grid1 = (B * P) // (NW * BQ)

    def _estream(w):
        return pl.BlockSpec((BQ, F_IN), lambda g, w=w: (g * NW + w, 0))

    s_flat = pl.pallas_call(
        _stage1_body,
        grid=(grid1,),
        in_specs=[
            _estream(0), _estream(1), _estream(2), _estream(3),
            pl.BlockSpec((NW * BQ, 8), lambda g: (g % (nrb // NW), 0)),
            pl.BlockSpec((F_IN, 12), lambda g: (0, 0)),
            pl.BlockSpec((8, 12), lambda g: (0, 0)),
            pl.BlockSpec((12, 128), lambda g: (0, 0)),
        ],
        out_specs=pl.BlockSpec((12, NW * BQ), lambda g: (0, g)),
        out_shape=jax.ShapeDtypeStruct((12, B * P), f32),
    )(emb2d, emb2d, emb2d, emb2d, refp, w_emb, w_refp, b12t)

    s3 = s_flat.reshape(12, B, P)
    sd = s3[0:3].transpose(2, 1, 0).reshape(P, 18)
    sr = s3[3:6].transpose(2, 1, 0).reshape(P, 18)
    s36 = jnp.concatenate([sd, sr], axis=1)
    ld = s3[6:9].transpose(2, 1, 0).reshape(P, 18)
    lr = s3[9:12].transpose(2, 1, 0).reshape(P, 18)
    sself = jnp.concatenate([ld, lr], axis=1)

    nb2 = P // BP2
    pm18, rgb18 = pl.pallas_call(
        _stage2_body,
        grid=(nb2,),
        in_specs=[
            pl.BlockSpec((BP2, P), lambda j: (j, 0)),
            pl.BlockSpec((P, 36), lambda j: (0, 0)),
            pl.BlockSpec((BP2, 36), lambda j: (j, 0)),
        ],
        out_specs=[
            pl.BlockSpec((BP2, 18), lambda j: (j, 0)),
            pl.BlockSpec((BP2, 18), lambda j: (j, 0)),
        ],
        out_shape=[
            jax.ShapeDtypeStruct((P, 18), f32),
            jax.ShapeDtypeStruct((P, 18), f32),
        ],
    )(adj.astype(f32), s36, sself)

    points_move = pm18.reshape(P, B, 3).transpose(1, 0, 2)
    rgb = rgb18.reshape(P, B, 3).transpose(1, 0, 2)
    return (points_move, rgb)
